# baseline (speedup 1.0000x reference)
import jax
import jax.numpy as jnp
from jax import lax
from jax.experimental import pallas as pl
from jax.experimental.pallas import tpu as pltpu

_C = 1024
_G = 64


def _body(bs_ref, cs_ref, vs_ref, yp_ref, yt_ref, out_ref, accr):
    g = pl.program_id(0)
    v = vs_ref[g]
    lane = lax.broadcasted_iota(jnp.int32, (1, 4, 4, _C), 3)
    msk = lane < v
    d = yp_ref[...] - yt_ref[...]
    contrib = jnp.where(msk, d * d, 0.0)

    @pl.when(g == 0)
    def _init():
        accr[...] = contrib

    @pl.when(g > 0)
    def _acc():
        accr[...] += contrib

    @pl.when(g == _G - 1)
    def _final():
        out_ref[0, 0] = jnp.sum(accr[...])


def kernel(y_pred, y_true, lengths):
    yp = jnp.transpose(y_pred, (0, 2, 3, 1))
    yt = jnp.transpose(y_true, (0, 2, 3, 1))
    thr = jnp.maximum(lengths.astype(jnp.int32) - 1, 0)

    nb = (thr + (_C - 1)) // _C
    cum = jnp.cumsum(nb)
    n = cum[-1]
    g = jnp.arange(_G, dtype=jnp.int32)
    gc = jnp.minimum(g, jnp.maximum(n - 1, 0))
    bs = jnp.searchsorted(cum, gc, side="right").astype(jnp.int32)
    bsc = jnp.minimum(bs, 15)
    start = cum[bsc] - nb[bsc]
    cs = gc - start
    v_full = jnp.minimum(thr[bsc] - cs * _C, _C)
    vs = jnp.where(g < n, v_full, 0).astype(jnp.int32)

    grid_spec = pltpu.PrefetchScalarGridSpec(
        num_scalar_prefetch=3,
        grid=(_G,),
        in_specs=[
            pl.BlockSpec((1, 4, 4, _C), lambda g, bs, cs, vs: (bs[g], 0, 0, cs[g])),
            pl.BlockSpec((1, 4, 4, _C), lambda g, bs, cs, vs: (bs[g], 0, 0, cs[g])),
        ],
        out_specs=pl.BlockSpec(memory_space=pltpu.SMEM),
        scratch_shapes=[
            pltpu.VMEM((1, 4, 4, _C), jnp.float32),
        ],
    )
    out = pl.pallas_call(
        _body,
        grid_spec=grid_spec,
        out_shape=jax.ShapeDtypeStruct((1, 1), jnp.float32),
    )(bsc, cs, vs, yp, yt)

    cnt = (jnp.sum(thr) * 16).astype(jnp.float32)
    return out[0, 0] / cnt


# speedup vs baseline: 1.8826x; 1.8826x over previous
import jax
import jax.numpy as jnp
from jax import lax
from jax.experimental import pallas as pl
from jax.experimental.pallas import tpu as pltpu

_C = 1024
_D = 8
_MAXN = 64


def _body(bs_ref, cs_ref, vs_ref, nn_ref, yp_ref, yt_ref, out_ref,
          bp, bt, accr, semp, semt):
    i32 = jnp.int32
    n = nn_ref[0]

    def _fire(idx, slot):
        b = bs_ref[idx]
        c = cs_ref[idx]
        src_p = yp_ref.at[b, :, :, pl.ds(c * _C, _C)]
        pltpu.make_async_copy(src_p, bp.at[slot], semp.at[slot]).start()
        src_t = yt_ref.at[b, :, :, pl.ds(c * _C, _C)]
        pltpu.make_async_copy(src_t, bt.at[slot], semt.at[slot]).start()

    def _drain(slot):
        dummy = yp_ref.at[0, :, :, pl.ds(0, _C)]
        pltpu.make_async_copy(dummy, bp.at[slot], semp.at[slot]).wait()
        pltpu.make_async_copy(dummy, bt.at[slot], semt.at[slot]).wait()

    accr[...] = jnp.zeros_like(accr)
    for slot in range(_D):
        @pl.when(slot < n)
        def _(slot=slot):
            _fire(i32(slot), slot)

    nouter = (n + (_D - 1)) >> 3

    def outer(it, _):
        base = it * _D
        for slot in range(_D):
            idx = base + slot
            live = idx < n

            @pl.when(live)
            def _(slot=slot):
                _drain(slot)

            v = jnp.where(live, vs_ref[jnp.minimum(idx, i32(_MAXN - 1))], 0)
            lane = lax.broadcasted_iota(i32, (4, 4, _C), 2)
            msk = lane < v
            d = bp[slot] - bt[slot]
            accr[...] += jnp.where(msk, d * d, 0.0)

            @pl.when(idx + _D < n)
            def _(slot=slot, idx=idx):
                _fire(idx + _D, slot)
        return 0

    lax.fori_loop(0, nouter, outer, 0)
    out_ref[0, 0] = jnp.sum(accr[...])


def kernel(y_pred, y_true, lengths):
    yp = jnp.transpose(y_pred, (0, 2, 3, 1))
    yt = jnp.transpose(y_true, (0, 2, 3, 1))
    thr = jnp.maximum(lengths.astype(jnp.int32) - 1, 0)

    nb = (thr + (_C - 1)) // _C
    cum = jnp.cumsum(nb)
    n = cum[-1]
    g = jnp.arange(_MAXN, dtype=jnp.int32)
    gc = jnp.minimum(g, jnp.maximum(n - 1, 0))
    bs = jnp.minimum(jnp.searchsorted(cum, gc, side="right").astype(jnp.int32), 15)
    cs = gc - (cum[bs] - nb[bs])
    vs = jnp.minimum(thr[bs] - cs * _C, _C).astype(jnp.int32)
    nn = jnp.full((1,), n, jnp.int32)

    grid_spec = pltpu.PrefetchScalarGridSpec(
        num_scalar_prefetch=4,
        grid=(1,),
        in_specs=[
            pl.BlockSpec(memory_space=pl.ANY),
            pl.BlockSpec(memory_space=pl.ANY),
        ],
        out_specs=pl.BlockSpec(memory_space=pltpu.SMEM),
        scratch_shapes=[
            pltpu.VMEM((_D, 4, 4, _C), jnp.float32),
            pltpu.VMEM((_D, 4, 4, _C), jnp.float32),
            pltpu.VMEM((4, 4, _C), jnp.float32),
            pltpu.SemaphoreType.DMA((_D,)),
            pltpu.SemaphoreType.DMA((_D,)),
        ],
    )
    out = pl.pallas_call(
        _body,
        grid_spec=grid_spec,
        out_shape=jax.ShapeDtypeStruct((1, 1), jnp.float32),
    )(bs, cs, vs, nn, yp, yt)

    cnt = (jnp.sum(thr) * 16).astype(jnp.float32)
    return out[0, 0] / cnt
